# Initial kernel scaffold; baseline (speedup 1.0000x reference)
import jax
import jax.numpy as jnp
from jax.experimental import pallas as pl


def kernel(x, threshold):
    raise NotImplementedError("write your pallas kernel here")


# speedup vs baseline: 1.9964x; 1.9964x over previous
import functools

import jax
import jax.numpy as jnp
from jax import lax
from jax.experimental import pallas as pl
from jax.experimental.pallas import tpu as pltpu
from jax.experimental.pallas import tpu_sc as plsc

_B = 16384
_NC = 2
_NS = 16
_NW = _NC * _NS
_PER_W = _B // _NW
_CHUNK = 2 * _PER_W
_TPAD = 1024
_L = 16


def _tec_body(xf_hbm, th_hbm, out_hbm, x_v, th_v, out_v):
    wid = lax.axis_index("s") * _NC + lax.axis_index("c")
    base = wid * _CHUNK
    pltpu.sync_copy(xf_hbm.at[pl.ds(base, _CHUNK)], x_v)
    pltpu.sync_copy(th_hbm, th_v)
    lane2 = lax.iota(jnp.int32, 16) * 2
    one = jnp.ones((_L,), jnp.int32)
    zero = jnp.zeros((_L,), jnp.int32)
    for j in range(_PER_W // _L):
        even = lane2 + (2 * _L * j)
        odd = even + 1
        x1 = plsc.load_gather(x_v, [even])
        x2 = plsc.load_gather(x_v, [odd])
        ti = x2.astype(jnp.int32)
        th = plsc.load_gather(th_v, [ti])
        bi = jnp.where(x1 < th, one, zero)
        plsc.store_scatter(out_v, [even], one - bi)
        plsc.store_scatter(out_v, [odd], bi)
    pltpu.sync_copy(out_v, out_hbm.at[pl.ds(base, _CHUNK)])


@functools.partial(jax.jit, static_argnums=())
def _sc_call(xf, th):
    mesh = plsc.VectorSubcoreMesh(core_axis_name="c", subcore_axis_name="s")
    return pl.kernel(
        _tec_body,
        out_type=jax.ShapeDtypeStruct((2 * _B,), jnp.int32),
        mesh=mesh,
        scratch_types=[
            pltpu.VMEM((_CHUNK,), jnp.float32),
            pltpu.VMEM((_TPAD,), jnp.float32),
            pltpu.VMEM((_CHUNK,), jnp.int32),
        ],
        compiler_params=pltpu.CompilerParams(needs_layout_passes=False),
    )(xf, th)


def kernel(x, threshold):
    xf = x.reshape(-1)
    th = jnp.pad(threshold, (0, _TPAD - threshold.shape[0]))
    out = _sc_call(xf, th)
    return out.reshape(_B, 2)


# speedup vs baseline: 2.0214x; 1.0125x over previous
import jax
import jax.numpy as jnp
from jax import lax
from jax.experimental import pallas as pl
from jax.experimental.pallas import tpu as pltpu
from jax.experimental.pallas import tpu_sc as plsc

_B = 16384
_NC = 2
_NS = 16
_NW = _NC * _NS
_PER_W = _B // _NW
_CHUNK = 2 * _PER_W
_T = 1000
_L = 16


def _tec_body(xf_hbm, th_hbm, out_hbm, x_v, th_v, out_v):
    wid = lax.axis_index("s") * _NC + lax.axis_index("c")
    base = wid * _CHUNK
    pltpu.sync_copy(xf_hbm.at[pl.ds(base, _CHUNK)], x_v)
    pltpu.sync_copy(th_hbm, th_v)
    lane2 = lax.iota(jnp.int32, 16) * 2
    one = jnp.ones((_L,), jnp.int32)
    zero = jnp.zeros((_L,), jnp.int32)
    for j in range(_PER_W // _L):
        even = lane2 + (2 * _L * j)
        odd = even + 1
        x1 = plsc.load_gather(x_v, [even])
        x2 = plsc.load_gather(x_v, [odd])
        ti = x2.astype(jnp.int32)
        th = plsc.load_gather(th_v, [ti])
        bi = jnp.where(x1 < th, one, zero)
        plsc.store_scatter(out_v, [even], one - bi)
        plsc.store_scatter(out_v, [odd], bi)
    pltpu.sync_copy(out_v, out_hbm.at[pl.ds(base, _CHUNK)])


def kernel(x, threshold):
    xf = x.reshape(-1)
    mesh = plsc.VectorSubcoreMesh(core_axis_name="c", subcore_axis_name="s")
    out = pl.kernel(
        _tec_body,
        out_type=jax.ShapeDtypeStruct((2 * _B,), jnp.int32),
        mesh=mesh,
        scratch_types=[
            pltpu.VMEM((_CHUNK,), jnp.float32),
            pltpu.VMEM((_T,), jnp.float32),
            pltpu.VMEM((_CHUNK,), jnp.int32),
        ],
        compiler_params=pltpu.CompilerParams(needs_layout_passes=False),
    )(xf, threshold)
    return out.reshape(_B, 2)


# speedup vs baseline: 4.4364x; 2.1947x over previous
import jax
import jax.numpy as jnp
from jax import lax
from jax.experimental import pallas as pl
from jax.experimental.pallas import tpu as pltpu
from jax.experimental.pallas import tpu_sc as plsc

_B = 16384
_NC = 2
_NS = 16
_NW = _NC * _NS
_PER_W = _B // _NW
_CHUNK = 2 * _PER_W
_BLK = 128
_T = 1000
_L = 16


def _tec_body(xf_hbm, th_hbm, out_hbm, x_v, th_v, out_v):
    wid = lax.axis_index("s") * _NC + lax.axis_index("c")
    base = wid * _CHUNK
    pltpu.sync_copy(xf_hbm.at[pl.ds(base, _CHUNK)], x_v)
    pltpu.sync_copy(th_hbm, th_v)
    one = jnp.ones((_L,), jnp.int32)
    zero = jnp.zeros((_L,), jnp.int32)
    for blk in range(_CHUNK // (2 * _BLK)):
        for k in range(_BLK // _L):
            o1 = 2 * _BLK * blk + _L * k
            o2 = o1 + _BLK
            x1 = x_v[pl.ds(o1, _L)]
            x2 = x_v[pl.ds(o2, _L)]
            ti = x2.astype(jnp.int32)
            th = plsc.load_gather(th_v, [ti])
            bi = jnp.where(x1 < th, one, zero)
            out_v[pl.ds(o1, _L)] = one - bi
            out_v[pl.ds(o2, _L)] = bi
    pltpu.sync_copy(out_v, out_hbm.at[pl.ds(base, _CHUNK)])


def kernel(x, threshold):
    xf = x.reshape(_B // _BLK, _BLK, 2).transpose(0, 2, 1).reshape(2 * _B)
    mesh = plsc.VectorSubcoreMesh(core_axis_name="c", subcore_axis_name="s")
    out = pl.kernel(
        _tec_body,
        out_type=jax.ShapeDtypeStruct((2 * _B,), jnp.int32),
        mesh=mesh,
        scratch_types=[
            pltpu.VMEM((_CHUNK,), jnp.float32),
            pltpu.VMEM((_T,), jnp.float32),
            pltpu.VMEM((_CHUNK,), jnp.int32),
        ],
        compiler_params=pltpu.CompilerParams(needs_layout_passes=False),
    )(xf, threshold)
    return out.reshape(_B // _BLK, 2, _BLK).transpose(0, 2, 1).reshape(_B, 2)


# speedup vs baseline: 4.6378x; 1.0454x over previous
import jax
import jax.numpy as jnp
from jax import lax
from jax.experimental import pallas as pl
from jax.experimental.pallas import tpu as pltpu
from jax.experimental.pallas import tpu_sc as plsc

_B = 16384
_NC = 2
_NS = 16
_NW = _NC * _NS
_PER_W = _B // _NW
_CHUNK = 2 * _PER_W
_BLK = 128
_T = 1000
_L = 16


def _tec_body(xf_hbm, th_hbm, out_hbm, x_v, th_v, out_v, sem_x, sem_t):
    wid = lax.axis_index("s") * _NC + lax.axis_index("c")
    base = wid * _CHUNK
    cp_x = pltpu.async_copy(xf_hbm.at[pl.ds(base, _CHUNK)], x_v, sem_x)
    cp_t = pltpu.async_copy(th_hbm, th_v, sem_t)
    cp_x.wait()
    cp_t.wait()
    one = jnp.ones((_L,), jnp.int32)
    zero = jnp.zeros((_L,), jnp.int32)

    def block(blk, carry):
        o0 = 2 * _BLK * blk
        for k in range(_BLK // _L):
            o1 = o0 + _L * k
            o2 = o1 + _BLK
            x1 = x_v[pl.ds(o1, _L)]
            x2 = x_v[pl.ds(o2, _L)]
            ti = x2.astype(jnp.int32)
            th = plsc.load_gather(th_v, [ti])
            bi = jnp.where(x1 < th, one, zero)
            out_v[pl.ds(o1, _L)] = one - bi
            out_v[pl.ds(o2, _L)] = bi
        return carry

    lax.fori_loop(0, _CHUNK // (2 * _BLK), block, 0)
    pltpu.sync_copy(out_v, out_hbm.at[pl.ds(base, _CHUNK)])


def kernel(x, threshold):
    xf = x.reshape(_B // _BLK, _BLK, 2).transpose(0, 2, 1).reshape(2 * _B)
    mesh = plsc.VectorSubcoreMesh(core_axis_name="c", subcore_axis_name="s")
    out = pl.kernel(
        _tec_body,
        out_type=jax.ShapeDtypeStruct((2 * _B,), jnp.int32),
        mesh=mesh,
        scratch_types=[
            pltpu.VMEM((_CHUNK,), jnp.float32),
            pltpu.VMEM((_T,), jnp.float32),
            pltpu.VMEM((_CHUNK,), jnp.int32),
            pltpu.SemaphoreType.DMA,
            pltpu.SemaphoreType.DMA,
        ],
        compiler_params=pltpu.CompilerParams(needs_layout_passes=False),
    )(xf, threshold)
    return out.reshape(_B // _BLK, 2, _BLK).transpose(0, 2, 1).reshape(_B, 2)


# speedup vs baseline: 5.0123x; 1.0807x over previous
import jax
import jax.numpy as jnp
from jax import lax
from jax.experimental import pallas as pl
from jax.experimental.pallas import tpu as pltpu
from jax.experimental.pallas import tpu_sc as plsc

_B = 16384
_NW = 16
_PER_W = _B // _NW
_CHUNK = 2 * _PER_W
_BLK = 128
_T = 1000
_L = 16


def _tec_body(xf_hbm, th_hbm, out_hbm, x_v, th_v, out_v, sem_x, sem_t):
    wid = lax.axis_index("s")
    base = wid * _CHUNK
    cp_x = pltpu.async_copy(xf_hbm.at[pl.ds(base, _CHUNK)], x_v, sem_x)
    cp_t = pltpu.async_copy(th_hbm, th_v, sem_t)
    cp_x.wait()
    cp_t.wait()
    one = jnp.ones((_L,), jnp.int32)
    zero = jnp.zeros((_L,), jnp.int32)

    def block(blk, carry):
        o0 = 2 * _BLK * blk
        for k in range(_BLK // _L):
            o1 = o0 + _L * k
            o2 = o1 + _BLK
            x1 = x_v[pl.ds(o1, _L)]
            x2 = x_v[pl.ds(o2, _L)]
            ti = x2.astype(jnp.int32)
            th = plsc.load_gather(th_v, [ti])
            bi = jnp.where(x1 < th, one, zero)
            out_v[pl.ds(o1, _L)] = one - bi
            out_v[pl.ds(o2, _L)] = bi
        return carry

    lax.fori_loop(0, _CHUNK // (2 * _BLK), block, 0)
    pltpu.sync_copy(out_v, out_hbm.at[pl.ds(base, _CHUNK)])


def kernel(x, threshold):
    xf = x.reshape(_B // _BLK, _BLK, 2).transpose(0, 2, 1).reshape(2 * _B)
    mesh = plsc.VectorSubcoreMesh(
        core_axis_name="c", subcore_axis_name="s", num_cores=1
    )
    out = pl.kernel(
        _tec_body,
        out_type=jax.ShapeDtypeStruct((2 * _B,), jnp.int32),
        mesh=mesh,
        scratch_types=[
            pltpu.VMEM((_CHUNK,), jnp.float32),
            pltpu.VMEM((_T,), jnp.float32),
            pltpu.VMEM((_CHUNK,), jnp.int32),
            pltpu.SemaphoreType.DMA,
            pltpu.SemaphoreType.DMA,
        ],
        compiler_params=pltpu.CompilerParams(needs_layout_passes=False),
    )(xf, threshold)
    return out.reshape(_B // _BLK, 2, _BLK).transpose(0, 2, 1).reshape(_B, 2)
